# baseline (speedup 1.0000x reference)
import math

import jax
import jax.numpy as jnp
from jax import lax
from jax.experimental import pallas as pl
from jax.experimental.pallas import tpu as pltpu
from jax.experimental.pallas import tpu_sc as plsc

N = 10000
E = 320000
D = 128
NUM_BRDFS = 4
NUM_BOUNCES = 8
FSM_GAMMA = 1e-3
SPEED_OF_SOUND = 343.0
MEAN_FREE_PATH = 5.0
AIR_ABS = 1e-3

WSCALE = math.exp(-AIR_ABS * MEAN_FREE_PATH) / 64.0
DECAY = math.exp(math.log(FSM_GAMMA) * (MEAN_FREE_PATH / SPEED_OF_SOUND))

NC = 2
NS = 16
L = 16
B = 96
NPAD = 10240
NHALF = NPAD // NC
RPT = NHALF // NS
RC = 64
RCH = RPT // RC
CAP = 172032
EPT = CAP // NS
NB = EPT // B


def _sc_body(x_hbm, row_hbm, col_hbm, attr_hbm, coef_hbm,
             out_hbm, r_hbm,
             racc, row_r, col_r, w_r, gbufs, abuf, coef_v, sg, ss):
    gb = gbufs
    c = lax.axis_index("c")
    s = lax.axis_index("s")
    gbase = c * NHALF + s * RPT
    lbase = s * RPT

    pltpu.sync_copy(coef_hbm, coef_v)
    pltpu.sync_copy(row_hbm.at[c, s], row_r)
    pltpu.sync_copy(col_hbm.at[c, s], col_r)

    cvec = coef_v[pl.ds(0, L)]
    c0 = cvec[0] * WSCALE
    c1 = cvec[1] * WSCALE
    c2 = cvec[2] * WSCALE
    c3 = cvec[3] * WSCALE
    coff = (c * NHALF).astype(jnp.int32)

    def init_batch(b, _):
        for k in range(NUM_BRDFS):
            pltpu.sync_copy(
                attr_hbm.at[pl.ds(((k * NC + c) * NS + s) * EPT + b * B, B)],
                abuf.at[k])
        for h in range(B // (2 * L)):
            lo = pl.ds(h * 2 * L, L)
            hi = pl.ds(h * 2 * L + L, L)
            wlo = (abuf[0, lo] * c0 + abuf[1, lo] * c1
                   + abuf[2, lo] * c2 + abuf[3, lo] * c3)
            whi = (abuf[0, hi] * c0 + abuf[1, hi] * c1
                   + abuf[2, hi] * c2 + abuf[3, hi] * c3)
            w_r[b, pl.ds(h * 2 * L, 2 * L)] = plsc.pack(
                wlo, whi, format=plsc.PackFormat.INTERLEAVED)
            col_r[b, lo] = col_r[b, lo] - coff
            col_r[b, hi] = col_r[b, hi] - coff
        return 0

    lax.fori_loop(0, NB, init_batch, 0, unroll=False)

    def init_rows(j, _):
        g0 = gb[0].at[pl.ds(0, RC)]
        g1 = gb[1].at[pl.ds(0, RC)]
        pltpu.sync_copy(x_hbm.at[pl.ds(gbase + j * RC, RC)], g0)
        pltpu.sync_copy(g0, r_hbm.at[pl.ds(gbase + j * RC, RC)])
        pltpu.sync_copy(g0, out_hbm.at[pl.ds(gbase + j * RC, RC)])

        def zz(e, _):
            for f in range(D // L):
                gb[1][e, pl.ds(f * L, L)] = jnp.zeros((L,), jnp.float32)
            return 0

        lax.fori_loop(0, RC, zz, 0, unroll=False)
        pltpu.sync_copy(g1, racc.at[pl.ds(lbase + j * RC, RC)])
        return 0

    lax.fori_loop(0, RCH, init_rows, 0, unroll=False)
    plsc.subcore_barrier()

    def bounce(t, _):
        pltpu.async_copy(racc.at[pl.ds(0, B)], gb[2], ss[2])
        pltpu.async_copy(racc.at[pl.ds(0, B)], gb[3], ss[3])
        pltpu.async_copy(r_hbm.at[row_r.at[0]], gb[0], sg[0])
        pltpu.async_copy(r_hbm.at[row_r.at[1]], gb[1], sg[1])

        def quad_body(hq, _):
            for p in range(4):
                j = hq * 4 + p
                q = (p + 2) % 4
                buf = gb[p]
                pltpu.make_async_copy(r_hbm.at[row_r.at[j]], buf,
                                      sg[p]).wait()

                def mult_h(h, _):
                    wlo, whi = plsc.unpack(
                        w_r[j, pl.ds(h * 2 * L, 2 * L)],
                        format=plsc.PackFormat.INTERLEAVED)
                    for k in range(2):
                        wv = (wlo, whi)[k]
                        for jj in range(L):
                            wj = jnp.full((L,), wv[jj], jnp.float32)
                            e = h * 2 * L + k * L + jj
                            for f in range(D // L):
                                sl = pl.ds(f * L, L)
                                buf[e, sl] = buf[e, sl] * wj
                    return 0

                lax.fori_loop(0, B // (2 * L), mult_h, 0, unroll=False)
                pltpu.async_copy(buf, racc.at[col_r.at[j]], ss[p], add=True)
                pltpu.make_async_copy(gb[q], racc.at[col_r.at[j]],
                                      ss[q]).wait()
                jn = jnp.minimum(j + 2, NB - 1)
                pltpu.async_copy(r_hbm.at[row_r.at[jn]], gb[q], sg[q])
            return 0

        lax.fori_loop(0, NB // 4, quad_body, 0, unroll=False)
        pltpu.make_async_copy(r_hbm.at[row_r.at[0]], gb[0], sg[0]).wait()
        pltpu.make_async_copy(r_hbm.at[row_r.at[0]], gb[1], sg[1]).wait()
        pltpu.make_async_copy(gb[2], racc.at[col_r.at[0]], ss[2]).wait()
        pltpu.make_async_copy(gb[3], racc.at[col_r.at[0]], ss[3]).wait()
        plsc.subcore_barrier()

        def chunk_body(j, _):
            g0 = gb[0].at[pl.ds(0, RC)]
            g1 = gb[1].at[pl.ds(0, RC)]
            pltpu.sync_copy(racc.at[pl.ds(lbase + j * RC, RC)], g0)
            pltpu.sync_copy(out_hbm.at[pl.ds(gbase + j * RC, RC)], g1)

            def row_body(e, _):
                for f in range(D // L):
                    sl = pl.ds(f * L, L)
                    v = gb[0][e, sl] * DECAY
                    gb[0][e, sl] = v
                    gb[1][e, sl] = gb[1][e, sl] + v
                return 0

            lax.fori_loop(0, RC, row_body, 0, unroll=False)
            pltpu.sync_copy(g0, r_hbm.at[pl.ds(gbase + j * RC, RC)])
            pltpu.sync_copy(g1, out_hbm.at[pl.ds(gbase + j * RC, RC)])

            def zz(e, _):
                for f in range(D // L):
                    gb[0][e, pl.ds(f * L, L)] = jnp.zeros((L,), jnp.float32)
                return 0

            lax.fori_loop(0, RC, zz, 0, unroll=False)
            pltpu.sync_copy(g0, racc.at[pl.ds(lbase + j * RC, RC)])
            return 0

        lax.fori_loop(0, RCH, chunk_body, 0, unroll=False)
        plsc.subcore_barrier()
        return 0

    lax.fori_loop(0, NUM_BOUNCES, bounce, 0, unroll=False)


def kernel(x, edge_index, edge_attr, brdf_coeffs):
    i32 = jnp.int32
    f32 = jnp.float32
    row = edge_index[0].astype(i32)
    col = edge_index[1].astype(i32)
    m1 = (col >= NHALF).astype(i32)
    cum1 = jnp.cumsum(m1)
    rank1 = cum1 - m1
    idx = jnp.arange(E, dtype=i32)
    rank0 = idx - rank1
    pos = jnp.where(m1 == 1, CAP + rank1, rank0)

    row2 = jnp.zeros(NC * CAP, i32).at[pos].set(row, unique_indices=True)
    cdef = (jnp.arange(NC * CAP, dtype=i32) // CAP) * NHALF
    col2 = cdef.at[pos].set(col, unique_indices=True)
    attr2 = jnp.zeros((NUM_BRDFS, NC * CAP), f32).at[:, pos].set(
        edge_attr.astype(f32).T, unique_indices=True)

    row3 = row2.reshape(NC, NS, NB, B)
    col3 = col2.reshape(NC, NS, NB, B)
    attr3 = attr2.reshape(NUM_BRDFS * NC * CAP)
    coef = jnp.pad(brdf_coeffs.astype(f32), (0, L - NUM_BRDFS))
    x2 = jnp.pad(x, ((0, NPAD - N), (0, 0)))

    mesh = plsc.VectorSubcoreMesh(core_axis_name="c", subcore_axis_name="s",
                                  num_cores=NC, num_subcores=NS)
    run = pl.kernel(
        _sc_body,
        out_type=(jax.ShapeDtypeStruct((NPAD, D), f32),
                  jax.ShapeDtypeStruct((NPAD, D), f32)),
        mesh=mesh,
        scratch_types=[
            pltpu.VMEM_SHARED((NHALF, D), f32),
            pltpu.VMEM((NB, B), i32),
            pltpu.VMEM((NB, B), i32),
            pltpu.VMEM((NB, B), jnp.bfloat16),
            [pltpu.VMEM((B, D), f32) for _ in range(4)],
            pltpu.VMEM((NUM_BRDFS, B), f32),
            pltpu.VMEM((L,), f32),
            [pltpu.SemaphoreType.DMA for _ in range(4)],
            [pltpu.SemaphoreType.DMA for _ in range(4)],
        ],
        compiler_params=pltpu.CompilerParams(use_tc_tiling_on_sc=False,
                                             needs_layout_passes=False),
    )
    out2, _ = run(x2, row3, col3, attr3, coef)
    return out2[:N]


# speedup vs baseline: 4.6356x; 4.6356x over previous
import math

import jax
import jax.numpy as jnp
from jax import lax
from jax.experimental import pallas as pl
from jax.experimental.pallas import tpu as pltpu
from jax.experimental.pallas import tpu_sc as plsc

N = 10000
E = 320000
D = 128
NUM_BRDFS = 4
NUM_BOUNCES = 8
FSM_GAMMA = 1e-3
SPEED_OF_SOUND = 343.0
MEAN_FREE_PATH = 5.0
AIR_ABS = 1e-3

WSCALE = math.exp(-AIR_ABS * MEAN_FREE_PATH) / 64.0
DECAY = math.exp(math.log(FSM_GAMMA) * (MEAN_FREE_PATH / SPEED_OF_SOUND))

NC = 2
NS = 16
L = 16
DH = D // NC
B = 128
NPAD = 10240
RPT = NPAD // NS
RCH = RPT // B
EPAD = 327680
EPT = EPAD // NS
NB = EPT // B
NROWS2 = NC * NPAD


def _sc_body(x_hbm, row_hbm, col_hbm, attr_hbm, coef_hbm,
             out_hbm, r_hbm,
             racc, row_r, col_r, w_r,
             g0, g1, g2, g3, abuf, coef_v,
             sg0, sg1, sg2, sg3, ss0, ss1, ss2, ss3):
    gb = (g0, g1, g2, g3)
    sg = (sg0, sg1, sg2, sg3)
    ss = (ss0, ss1, ss2, ss3)
    c = lax.axis_index("c")
    s = lax.axis_index("s")
    rbase = c * NPAD + s * RPT
    lbase = s * RPT

    pltpu.sync_copy(coef_hbm, coef_v)
    pltpu.sync_copy(row_hbm.at[s], row_r)
    pltpu.sync_copy(col_hbm.at[s], col_r)

    cvec = coef_v[pl.ds(0, L)]
    c0 = cvec[0] * WSCALE
    c1 = cvec[1] * WSCALE
    c2 = cvec[2] * WSCALE
    c3 = cvec[3] * WSCALE
    coff = (c * NPAD).astype(jnp.int32)

    def init_batch(b, _):
        for k in range(NUM_BRDFS):
            pltpu.sync_copy(attr_hbm.at[pl.ds(k * EPAD + s * EPT + b * B, B)],
                            abuf.at[k])
        for h in range(B // (2 * L)):
            lo = pl.ds(h * 2 * L, L)
            hi = pl.ds(h * 2 * L + L, L)
            wlo = (abuf[0, lo] * c0 + abuf[1, lo] * c1
                   + abuf[2, lo] * c2 + abuf[3, lo] * c3)
            whi = (abuf[0, hi] * c0 + abuf[1, hi] * c1
                   + abuf[2, hi] * c2 + abuf[3, hi] * c3)
            w_r[b, pl.ds(h * 2 * L, 2 * L)] = plsc.pack(
                wlo, whi, format=plsc.PackFormat.INTERLEAVED)
            row_r[b, lo] = row_r[b, lo] + coff
            row_r[b, hi] = row_r[b, hi] + coff
        return 0

    lax.fori_loop(0, NB, init_batch, 0, unroll=False)

    def init_rows(j, _):
        pltpu.sync_copy(x_hbm.at[pl.ds(rbase + j * B, B)], g0)
        pltpu.sync_copy(g0, r_hbm.at[pl.ds(rbase + j * B, B)])
        pltpu.sync_copy(g0, out_hbm.at[pl.ds(rbase + j * B, B)])

        def zz(e, _):
            for f in range(DH // L):
                g0[e, pl.ds(f * L, L)] = jnp.zeros((L,), jnp.float32)
            return 0

        lax.fori_loop(0, B, zz, 0, unroll=False)
        pltpu.sync_copy(g0, racc.at[pl.ds(lbase + j * B, B)])
        return 0

    lax.fori_loop(0, RCH, init_rows, 0, unroll=False)
    plsc.subcore_barrier()

    def bounce(t, _):
        pltpu.async_copy(racc.at[pl.ds(0, B)], gb[2], ss[2])
        pltpu.async_copy(racc.at[pl.ds(0, B)], gb[3], ss[3])
        pltpu.async_copy(r_hbm.at[row_r.at[0]], gb[0], sg[0])
        pltpu.async_copy(r_hbm.at[row_r.at[1]], gb[1], sg[1])

        def quad_body(hq, _):
            for p in range(4):
                j = hq * 4 + p
                q = (p + 2) % 4
                buf = gb[p]
                pltpu.make_async_copy(r_hbm.at[row_r.at[j]], buf,
                                      sg[p]).wait()

                def mult_h(h, _):
                    wlo, whi = plsc.unpack(
                        w_r[j, pl.ds(h * 2 * L, 2 * L)],
                        format=plsc.PackFormat.INTERLEAVED)
                    for k in range(2):
                        wv = (wlo, whi)[k]
                        for jj in range(L):
                            wj = jnp.full((L,), wv[jj], jnp.float32)
                            e = h * 2 * L + k * L + jj
                            for f in range(DH // L):
                                sl = pl.ds(f * L, L)
                                buf[e, sl] = buf[e, sl] * wj
                    return 0

                lax.fori_loop(0, B // (2 * L), mult_h, 0, unroll=False)
                pltpu.async_copy(buf, racc.at[col_r.at[j]], ss[p], add=True)
                pltpu.make_async_copy(gb[q], racc.at[col_r.at[j]],
                                      ss[q]).wait()
                jn = jnp.minimum(j + 2, NB - 1)
                pltpu.async_copy(r_hbm.at[row_r.at[jn]], gb[q], sg[q])
            return 0

        lax.fori_loop(0, NB // 4, quad_body, 0, unroll=False)
        pltpu.make_async_copy(r_hbm.at[row_r.at[0]], gb[0], sg[0]).wait()
        pltpu.make_async_copy(r_hbm.at[row_r.at[0]], gb[1], sg[1]).wait()
        pltpu.make_async_copy(gb[2], racc.at[col_r.at[0]], ss[2]).wait()
        pltpu.make_async_copy(gb[3], racc.at[col_r.at[0]], ss[3]).wait()
        plsc.subcore_barrier()

        def chunk_body(j, _):
            pltpu.sync_copy(racc.at[pl.ds(lbase + j * B, B)], g0)
            pltpu.sync_copy(out_hbm.at[pl.ds(rbase + j * B, B)], g1)

            def row_body(e, _):
                for f in range(DH // L):
                    sl = pl.ds(f * L, L)
                    v = g0[e, sl] * DECAY
                    g0[e, sl] = v
                    g1[e, sl] = g1[e, sl] + v
                return 0

            lax.fori_loop(0, B, row_body, 0, unroll=False)
            pltpu.sync_copy(g0, r_hbm.at[pl.ds(rbase + j * B, B)])
            pltpu.sync_copy(g1, out_hbm.at[pl.ds(rbase + j * B, B)])

            def zz(e, _):
                for f in range(DH // L):
                    g0[e, pl.ds(f * L, L)] = jnp.zeros((L,), jnp.float32)
                return 0

            lax.fori_loop(0, B, zz, 0, unroll=False)
            pltpu.sync_copy(g0, racc.at[pl.ds(lbase + j * B, B)])
            return 0

        lax.fori_loop(0, RCH, chunk_body, 0, unroll=False)
        plsc.subcore_barrier()
        return 0

    lax.fori_loop(0, NUM_BOUNCES, bounce, 0, unroll=False)


def kernel(x, edge_index, edge_attr, brdf_coeffs):
    x2 = x.reshape(N, NC, DH).transpose(1, 0, 2)
    x2 = jnp.pad(x2, ((0, 0), (0, NPAD - N), (0, 0))).reshape(NROWS2, DH)

    row = jnp.pad(edge_index[0].astype(jnp.int32), (0, EPAD - E))
    col = jnp.pad(edge_index[1].astype(jnp.int32), (0, EPAD - E))
    row2 = row.reshape(NS, NB, B)
    col2 = col.reshape(NS, NB, B)
    attr2 = jnp.pad(edge_attr.astype(jnp.float32).T,
                    ((0, 0), (0, EPAD - E))).reshape(NUM_BRDFS * EPAD)
    coef = jnp.pad(brdf_coeffs.astype(jnp.float32), (0, L - NUM_BRDFS))

    mesh = plsc.VectorSubcoreMesh(core_axis_name="c", subcore_axis_name="s",
                                  num_cores=NC, num_subcores=NS)
    f32 = jnp.float32
    run = pl.kernel(
        _sc_body,
        out_type=(jax.ShapeDtypeStruct((NROWS2, DH), f32),
                  jax.ShapeDtypeStruct((NROWS2, DH), f32)),
        mesh=mesh,
        scratch_types=[
            pltpu.VMEM_SHARED((NPAD, DH), f32),
            pltpu.VMEM((NB, B), jnp.int32),
            pltpu.VMEM((NB, B), jnp.int32),
            pltpu.VMEM((NB, B), jnp.bfloat16),
            pltpu.VMEM((B, DH), f32),
            pltpu.VMEM((B, DH), f32),
            pltpu.VMEM((B, DH), f32),
            pltpu.VMEM((B, DH), f32),
            pltpu.VMEM((NUM_BRDFS, B), f32),
            pltpu.VMEM((L,), f32),
            pltpu.SemaphoreType.DMA,
            pltpu.SemaphoreType.DMA,
            pltpu.SemaphoreType.DMA,
            pltpu.SemaphoreType.DMA,
            pltpu.SemaphoreType.DMA,
            pltpu.SemaphoreType.DMA,
            pltpu.SemaphoreType.DMA,
            pltpu.SemaphoreType.DMA,
        ],
        compiler_params=pltpu.CompilerParams(use_tc_tiling_on_sc=False,
                                             needs_layout_passes=False),
    )
    out2, _ = run(x2, row2, col2, attr2, coef)
    out = out2.reshape(NC, NPAD, DH)[:, :N]
    return out.transpose(1, 0, 2).reshape(N, D)
